# baseline (speedup 1.0000x reference)
import functools

import jax
import jax.numpy as jnp
from jax import lax
from jax.experimental import pallas as pl
from jax.experimental.pallas import tpu as pltpu
from jax.experimental.pallas import tpu_sc as plsc

D = 768
NW = 32
CH = 16
NSLOTS = 4
BT = 256


def _make_gather_kernel(total):
    mesh = plsc.VectorSubcoreMesh(core_axis_name="c", subcore_axis_name="s")
    tok_per_tile = total // NW
    n_chunks = tok_per_tile // CH

    @functools.partial(
        pl.kernel,
        mesh=mesh,
        out_type=(
            jax.ShapeDtypeStruct((total, D), jnp.float32),
            jax.ShapeDtypeStruct((total, D), jnp.float32),
        ),
        scratch_types=(
            [pltpu.VMEM((tok_per_tile,), jnp.int32)]
            + [pltpu.VMEM((CH, D), jnp.float32)] * (2 * NSLOTS)
            + [pltpu.SemaphoreType.DMA] * (4 * NSLOTS)
        ),
    )
    def gather_kernel(ids_hbm, wr_hbm, wi_hbm, outr_hbm, outi_hbm,
                      ids_v, *rest):
        bufr = rest[0:NSLOTS]
        bufi = rest[NSLOTS:2 * NSLOTS]
        semgr = rest[2 * NSLOTS:3 * NSLOTS]
        semgi = rest[3 * NSLOTS:4 * NSLOTS]
        semor = rest[4 * NSLOTS:5 * NSLOTS]
        semoi = rest[5 * NSLOTS:6 * NSLOTS]

        cid = lax.axis_index("c")
        sid = lax.axis_index("s")
        wid = sid * 2 + cid
        tok0 = wid * tok_per_tile

        pltpu.sync_copy(ids_hbm.at[pl.ds(tok0, tok_per_tile)], ids_v)

        def start_gather(g):
            slot = g % NSLOTS
            ic = ids_v.at[pl.ds(g * CH, CH)]
            return (
                pltpu.async_copy(wr_hbm.at[ic], bufr[slot], semgr[slot]),
                pltpu.async_copy(wi_hbm.at[ic], bufi[slot], semgi[slot]),
            )

        gath = {}
        outs = {}
        for g in range(min(NSLOTS, n_chunks)):
            gath[g] = start_gather(g)

        for g in range(n_chunks):
            slot = g % NSLOTS
            for cp in gath.pop(g):
                cp.wait()
            dst = pl.ds(tok0 + g * CH, CH)
            outs[g] = (
                pltpu.async_copy(bufr[slot], outr_hbm.at[dst], semor[slot]),
                pltpu.async_copy(bufi[slot], outi_hbm.at[dst], semoi[slot]),
            )
            ng = g + NSLOTS
            if ng < n_chunks:
                for cp in outs.pop(g):
                    cp.wait()
                gath[ng] = start_gather(ng)

        for g in sorted(outs):
            for cp in outs[g]:
                cp.wait()

    return gather_kernel


def _rotate_body(theta_ref, gr_ref, gi_ref, pos_ref, out_ref):
    th = theta_ref[...]
    c = jnp.cos(th)
    s = jnp.sin(th)
    x = gr_ref[...] + pos_ref[...]
    y = gi_ref[...]
    out_ref[0, 0] = x * c - y * s
    out_ref[1, 0] = x * s + y * c


def _rotate_body_aliased(theta_ref, gr_ref, gi_ref, pos_ref, _prev_ref,
                         out_ref):
    _rotate_body(theta_ref, gr_ref, gi_ref, pos_ref, out_ref)


def _rotate_chunk(theta_col, gr, gi, pos_table, b, bsz, seq, prev):
    nb = seq // BT
    in_specs = [
        pl.BlockSpec((BT, 1), lambda i: (i, 0)),
        pl.BlockSpec((BT, D), lambda i: (i, 0)),
        pl.BlockSpec((BT, D), lambda i: (i, 0)),
        pl.BlockSpec((BT, D), lambda i: (i, 0)),
    ]
    operands = [theta_col, gr, gi, pos_table]
    body = _rotate_body
    aliases = {}
    if prev is not None:
        in_specs.append(pl.BlockSpec(memory_space=pl.ANY))
        operands.append(prev)
        body = _rotate_body_aliased
        aliases = {4: 0}
    return pl.pallas_call(
        body,
        grid=(nb,),
        in_specs=in_specs,
        out_specs=pl.BlockSpec((2, 1, BT, D), lambda i, b=b: (0, b, i, 0)),
        out_shape=jax.ShapeDtypeStruct((2, bsz, seq, D), jnp.float32),
        input_output_aliases=aliases,
    )(*operands)


def kernel(input_ids, initial_phase, W_real, W_imag, pos_table):
    bsz, seq = input_ids.shape
    ids = input_ids.astype(jnp.int32)
    theta = initial_phase
    gathered = []
    sc_gather = _make_gather_kernel(seq)
    for b in range(bsz):
        gathered.append(sc_gather(ids[b], W_real, W_imag))
    out = None
    for b in range(bsz):
        gr, gi = gathered[b]
        out = _rotate_chunk(theta[b].reshape(seq, 1), gr, gi, pos_table,
                            b, bsz, seq, out)
    return out


# speedup vs baseline: 1.0414x; 1.0414x over previous
import functools

import jax
import jax.numpy as jnp
from jax import lax
from jax.experimental import pallas as pl
from jax.experimental.pallas import tpu as pltpu
from jax.experimental.pallas import tpu_sc as plsc

D = 768
NW = 32
CH = 16
NSLOTS = 4
BT = 256
NCHUNK = 4


def _make_gather_kernel(total):
    mesh = plsc.VectorSubcoreMesh(core_axis_name="c", subcore_axis_name="s")
    tok_per_tile = total // NW
    n_chunks = tok_per_tile // CH

    @functools.partial(
        pl.kernel,
        mesh=mesh,
        out_type=(
            jax.ShapeDtypeStruct((total, D), jnp.float32),
            jax.ShapeDtypeStruct((total, D), jnp.float32),
        ),
        scratch_types=(
            [pltpu.VMEM((tok_per_tile,), jnp.int32)]
            + [pltpu.VMEM((CH, D), jnp.float32)] * (2 * NSLOTS)
            + [pltpu.SemaphoreType.DMA] * (4 * NSLOTS)
        ),
    )
    def gather_kernel(ids_hbm, wr_hbm, wi_hbm, outr_hbm, outi_hbm,
                      ids_v, *rest):
        bufr = rest[0:NSLOTS]
        bufi = rest[NSLOTS:2 * NSLOTS]
        semgr = rest[2 * NSLOTS:3 * NSLOTS]
        semgi = rest[3 * NSLOTS:4 * NSLOTS]
        semor = rest[4 * NSLOTS:5 * NSLOTS]
        semoi = rest[5 * NSLOTS:6 * NSLOTS]

        cid = lax.axis_index("c")
        sid = lax.axis_index("s")
        wid = sid * 2 + cid
        tok0 = wid * tok_per_tile

        pltpu.sync_copy(ids_hbm.at[pl.ds(tok0, tok_per_tile)], ids_v)

        def start_gather(g):
            slot = g % NSLOTS
            ic = ids_v.at[pl.ds(g * CH, CH)]
            return (
                pltpu.async_copy(wr_hbm.at[ic], bufr[slot], semgr[slot]),
                pltpu.async_copy(wi_hbm.at[ic], bufi[slot], semgi[slot]),
            )

        gath = {}
        outs = {}
        for g in range(min(NSLOTS, n_chunks)):
            gath[g] = start_gather(g)

        for g in range(n_chunks):
            slot = g % NSLOTS
            for cp in gath.pop(g):
                cp.wait()
            dst = pl.ds(tok0 + g * CH, CH)
            outs[g] = (
                pltpu.async_copy(bufr[slot], outr_hbm.at[dst], semor[slot]),
                pltpu.async_copy(bufi[slot], outi_hbm.at[dst], semoi[slot]),
            )
            ng = g + NSLOTS
            if ng < n_chunks:
                for cp in outs.pop(g):
                    cp.wait()
                gath[ng] = start_gather(ng)

        for g in sorted(outs):
            for cp in outs[g]:
                cp.wait()

    return gather_kernel


def _rotate_body(theta_ref, gr_ref, gi_ref, pos_ref, out_ref):
    th = theta_ref[...]
    c = jnp.cos(th)
    s = jnp.sin(th)
    x = gr_ref[...] + pos_ref[...]
    y = gi_ref[...]
    out_ref[0, 0] = x * c - y * s
    out_ref[1, 0] = x * s + y * c


def _rotate_body_aliased(theta_ref, gr_ref, gi_ref, pos_ref, _prev_ref,
                         out_ref):
    _rotate_body(theta_ref, gr_ref, gi_ref, pos_ref, out_ref)


def _rotate_chunk(theta_col, gr, gi, pos_table, c, span, bsz, seq, prev):
    nbc = span // BT
    in_specs = [
        pl.BlockSpec((BT, 1), lambda i, b: (b * nbc + i, 0)),
        pl.BlockSpec((BT, D), lambda i, b: (b * nbc + i, 0)),
        pl.BlockSpec((BT, D), lambda i, b: (b * nbc + i, 0)),
        pl.BlockSpec((BT, D), lambda i, b, c=c: (c * nbc + i, 0)),
    ]
    operands = [theta_col, gr, gi, pos_table]
    body = _rotate_body
    aliases = {}
    if prev is not None:
        in_specs.append(pl.BlockSpec(memory_space=pl.ANY))
        operands.append(prev)
        body = _rotate_body_aliased
        aliases = {4: 0}
    return pl.pallas_call(
        body,
        grid=(nbc, bsz),
        in_specs=in_specs,
        out_specs=pl.BlockSpec(
            (2, 1, BT, D), lambda i, b, c=c: (0, b, c * nbc + i, 0)),
        out_shape=jax.ShapeDtypeStruct((2, bsz, seq, D), jnp.float32),
        input_output_aliases=aliases,
    )(*operands)


def kernel(input_ids, initial_phase, W_real, W_imag, pos_table):
    bsz, seq = input_ids.shape
    span = seq // NCHUNK
    ids = input_ids.astype(jnp.int32)
    theta = initial_phase
    sc_gather = _make_gather_kernel(bsz * span)
    gathered = []
    for c in range(NCHUNK):
        ids_c = ids[:, c * span:(c + 1) * span].reshape(bsz * span)
        gathered.append(sc_gather(ids_c, W_real, W_imag))
    out = None
    for c in range(NCHUNK):
        gr, gi = gathered[c]
        th_c = theta[:, c * span:(c + 1) * span].reshape(bsz * span, 1)
        out = _rotate_chunk(th_c, gr, gi, pos_table, c, span, bsz, seq, out)
    return out


# speedup vs baseline: 1.0822x; 1.0391x over previous
import functools

import jax
import jax.numpy as jnp
from jax import lax
from jax.experimental import pallas as pl
from jax.experimental.pallas import tpu as pltpu
from jax.experimental.pallas import tpu_sc as plsc

D = 768
NW = 32
CH = 16
NSLOTS = 4
NCHUNK = 4


def _make_gather_kernel(total, base):
    mesh = plsc.VectorSubcoreMesh(core_axis_name="c", subcore_axis_name="s")
    tok_per_tile = total // NW
    n_chunks = tok_per_tile // CH

    @functools.partial(
        pl.kernel,
        mesh=mesh,
        out_type=jax.ShapeDtypeStruct((2, total, D), jnp.float32),
        scratch_types=(
            [pltpu.VMEM((tok_per_tile,), jnp.int32)]
            + [pltpu.VMEM((CH, D), jnp.float32)] * (2 * NSLOTS)
            + [pltpu.SemaphoreType.DMA] * (4 * NSLOTS)
        ),
    )
    def gather_kernel(ids_hbm, wr_hbm, wi_hbm, out_hbm, ids_v, *rest):
        bufr = rest[0:NSLOTS]
        bufi = rest[NSLOTS:2 * NSLOTS]
        semgr = rest[2 * NSLOTS:3 * NSLOTS]
        semgi = rest[3 * NSLOTS:4 * NSLOTS]
        semor = rest[4 * NSLOTS:5 * NSLOTS]
        semoi = rest[5 * NSLOTS:6 * NSLOTS]

        cid = lax.axis_index("c")
        sid = lax.axis_index("s")
        wid = sid * 2 + cid
        tok0 = wid * tok_per_tile

        pltpu.sync_copy(ids_hbm.at[pl.ds(base + tok0, tok_per_tile)], ids_v)

        def start_gather(g):
            slot = g % NSLOTS
            ic = ids_v.at[pl.ds(g * CH, CH)]
            return (
                pltpu.async_copy(wr_hbm.at[ic], bufr[slot], semgr[slot]),
                pltpu.async_copy(wi_hbm.at[ic], bufi[slot], semgi[slot]),
            )

        gath = {}
        outs = {}
        for g in range(min(NSLOTS, n_chunks)):
            gath[g] = start_gather(g)

        for g in range(n_chunks):
            slot = g % NSLOTS
            for cp in gath.pop(g):
                cp.wait()
            dst = pl.ds(tok0 + g * CH, CH)
            outs[g] = (
                pltpu.async_copy(bufr[slot], out_hbm.at[0, dst], semor[slot]),
                pltpu.async_copy(bufi[slot], out_hbm.at[1, dst], semoi[slot]),
            )
            ng = g + NSLOTS
            if ng < n_chunks:
                for cp in outs.pop(g):
                    cp.wait()
                gath[ng] = start_gather(ng)

        for g in sorted(outs):
            for cp in outs[g]:
                cp.wait()

    return gather_kernel


def _rotate_body(theta_ref, g_ref, pos_ref, out_ref):
    th = theta_ref[...]
    c = jnp.cos(th)
    s = jnp.sin(th)
    x = g_ref[0] + pos_ref[...]
    y = g_ref[1]
    out_ref[0, 0] = x * c - y * s
    out_ref[1, 0] = x * s + y * c


def _rotate_body_aliased(theta_ref, g_ref, pos_ref, _prev_ref, out_ref):
    _rotate_body(theta_ref, g_ref, pos_ref, out_ref)


def _rotate_chunk(theta_t, g, pos_table, c, span, bsz, seq, prev):
    in_specs = [
        pl.BlockSpec((span, 1), lambda b, c=c, bsz=bsz: (c * bsz + b, 0)),
        pl.BlockSpec((2, span, D), lambda b: (0, b, 0)),
        pl.BlockSpec((span, D), lambda b, c=c: (c, 0)),
    ]
    operands = [theta_t, g, pos_table]
    body = _rotate_body
    aliases = {}
    if prev is not None:
        in_specs.append(pl.BlockSpec(memory_space=pl.ANY))
        operands.append(prev)
        body = _rotate_body_aliased
        aliases = {3: 0}
    return pl.pallas_call(
        body,
        grid=(bsz,),
        in_specs=in_specs,
        out_specs=pl.BlockSpec(
            (2, 1, span, D), lambda b, c=c: (0, b, c, 0)),
        out_shape=jax.ShapeDtypeStruct((2, bsz, seq, D), jnp.float32),
        input_output_aliases=aliases,
    )(*operands)


def kernel(input_ids, initial_phase, W_real, W_imag, pos_table):
    bsz, seq = input_ids.shape
    span = seq // NCHUNK
    total = bsz * span
    ids_t = (input_ids.astype(jnp.int32)
             .reshape(bsz, NCHUNK, span)
             .transpose(1, 0, 2)
             .reshape(NCHUNK * total))
    theta_t = (initial_phase
               .reshape(bsz, NCHUNK, span)
               .transpose(1, 0, 2)
               .reshape(NCHUNK * total, 1))
    gathered = [
        _make_gather_kernel(total, c * total)(ids_t, W_real, W_imag)
        for c in range(NCHUNK)
    ]
    out = None
    for c in range(NCHUNK):
        out = _rotate_chunk(theta_t, gathered[c], pos_table, c, span, bsz,
                            seq, out)
    return out


# speedup vs baseline: 1.0919x; 1.0090x over previous
import functools

import jax
import jax.numpy as jnp
from jax import lax
from jax.experimental import pallas as pl
from jax.experimental.pallas import tpu as pltpu
from jax.experimental.pallas import tpu_sc as plsc

D = 768
NW = 32
CH = 16
NSLOTS = 4
NCHUNK = 4


def _make_gather_kernel(bsz, seq, span, chunk):
    mesh = plsc.VectorSubcoreMesh(core_axis_name="c", subcore_axis_name="s")
    total = bsz * span
    tok_per_tile = total // NW
    n_chunks = tok_per_tile // CH

    @functools.partial(
        pl.kernel,
        mesh=mesh,
        out_type=jax.ShapeDtypeStruct((2, total, D), jnp.float32),
        scratch_types=(
            [pltpu.VMEM((tok_per_tile,), jnp.int32)]
            + [pltpu.VMEM((CH, D), jnp.float32)] * (2 * NSLOTS)
            + [pltpu.SemaphoreType.DMA] * (4 * NSLOTS)
        ),
    )
    def gather_kernel(ids_hbm, wr_hbm, wi_hbm, out_hbm, ids_v, *rest):
        bufr = rest[0:NSLOTS]
        bufi = rest[NSLOTS:2 * NSLOTS]
        semgr = rest[2 * NSLOTS:3 * NSLOTS]
        semgi = rest[3 * NSLOTS:4 * NSLOTS]
        semor = rest[4 * NSLOTS:5 * NSLOTS]
        semoi = rest[5 * NSLOTS:6 * NSLOTS]

        cid = lax.axis_index("c")
        sid = lax.axis_index("s")
        wid = sid * 2 + cid
        tok0 = wid * tok_per_tile
        b = tok0 // span
        j0 = tok0 % span
        src0 = b * seq + chunk * span + j0

        pltpu.sync_copy(ids_hbm.at[pl.ds(src0, tok_per_tile)], ids_v)

        def start_gather(g):
            slot = g % NSLOTS
            ic = ids_v.at[pl.ds(g * CH, CH)]
            return (
                pltpu.async_copy(wr_hbm.at[ic], bufr[slot], semgr[slot]),
                pltpu.async_copy(wi_hbm.at[ic], bufi[slot], semgi[slot]),
            )

        gath = {}
        outs = {}
        for g in range(min(NSLOTS, n_chunks)):
            gath[g] = start_gather(g)

        for g in range(n_chunks):
            slot = g % NSLOTS
            for cp in gath.pop(g):
                cp.wait()
            dst = pl.ds(tok0 + g * CH, CH)
            outs[g] = (
                pltpu.async_copy(bufr[slot], out_hbm.at[0, dst], semor[slot]),
                pltpu.async_copy(bufi[slot], out_hbm.at[1, dst], semoi[slot]),
            )
            ng = g + NSLOTS
            if ng < n_chunks:
                for cp in outs.pop(g):
                    cp.wait()
                gath[ng] = start_gather(ng)

        for g in sorted(outs):
            for cp in outs[g]:
                cp.wait()

    return gather_kernel


def _rotate_body(theta_ref, g_ref, pos_ref, out_ref):
    b = pl.program_id(0)
    th = jnp.transpose(theta_ref[pl.ds(b, 1), :])
    c = jnp.cos(th)
    s = jnp.sin(th)
    x = g_ref[0] + pos_ref[...]
    y = g_ref[1]
    out_ref[0, 0] = x * c - y * s
    out_ref[1, 0] = x * s + y * c


def _rotate_body_aliased(theta_ref, g_ref, pos_ref, _prev_ref, out_ref):
    _rotate_body(theta_ref, g_ref, pos_ref, out_ref)


def _rotate_chunk(theta, g, pos_table, c, span, bsz, seq, prev):
    in_specs = [
        pl.BlockSpec((bsz, span), lambda b, c=c: (0, c)),
        pl.BlockSpec((2, span, D), lambda b: (0, b, 0)),
        pl.BlockSpec((span, D), lambda b, c=c: (c, 0)),
    ]
    operands = [theta, g, pos_table]
    body = _rotate_body
    aliases = {}
    if prev is not None:
        in_specs.append(pl.BlockSpec(memory_space=pl.ANY))
        operands.append(prev)
        body = _rotate_body_aliased
        aliases = {3: 0}
    return pl.pallas_call(
        body,
        grid=(bsz,),
        in_specs=in_specs,
        out_specs=pl.BlockSpec(
            (2, 1, span, D), lambda b, c=c: (0, b, c, 0)),
        out_shape=jax.ShapeDtypeStruct((2, bsz, seq, D), jnp.float32),
        input_output_aliases=aliases,
    )(*operands)


def kernel(input_ids, initial_phase, W_real, W_imag, pos_table):
    bsz, seq = input_ids.shape
    span = seq // NCHUNK
    ids_flat = input_ids.astype(jnp.int32).reshape(bsz * seq)
    gathered = [
        _make_gather_kernel(bsz, seq, span, c)(ids_flat, W_real, W_imag)
        for c in range(NCHUNK)
    ]
    out = None
    for c in range(NCHUNK):
        out = _rotate_chunk(initial_phase, gathered[c], pos_table, c, span,
                            bsz, seq, out)
    return out


# speedup vs baseline: 1.1449x; 1.0485x over previous
import functools

import jax
import jax.numpy as jnp
from jax import lax
from jax.experimental import pallas as pl
from jax.experimental.pallas import tpu as pltpu
from jax.experimental.pallas import tpu_sc as plsc

D = 768
NW = 32
CH = 16
NSLOTS = 4
NCHUNK = 2


def _make_gather_kernel(bsz, seq, span, chunk):
    mesh = plsc.VectorSubcoreMesh(core_axis_name="c", subcore_axis_name="s")
    total = bsz * span
    tok_per_tile = total // NW
    n_chunks = tok_per_tile // CH

    @functools.partial(
        pl.kernel,
        mesh=mesh,
        out_type=jax.ShapeDtypeStruct((2, total, D), jnp.float32),
        scratch_types=(
            [pltpu.VMEM((tok_per_tile,), jnp.int32)]
            + [pltpu.VMEM((CH, D), jnp.float32)] * (2 * NSLOTS)
            + [pltpu.SemaphoreType.DMA] * (4 * NSLOTS)
        ),
    )
    def gather_kernel(ids_hbm, wr_hbm, wi_hbm, out_hbm, ids_v, *rest):
        bufr = rest[0:NSLOTS]
        bufi = rest[NSLOTS:2 * NSLOTS]
        semgr = rest[2 * NSLOTS:3 * NSLOTS]
        semgi = rest[3 * NSLOTS:4 * NSLOTS]
        semor = rest[4 * NSLOTS:5 * NSLOTS]
        semoi = rest[5 * NSLOTS:6 * NSLOTS]

        cid = lax.axis_index("c")
        sid = lax.axis_index("s")
        wid = sid * 2 + cid
        tok0 = wid * tok_per_tile
        b = tok0 // span
        j0 = tok0 % span
        src0 = b * seq + chunk * span + j0

        pltpu.sync_copy(ids_hbm.at[pl.ds(src0, tok_per_tile)], ids_v)

        def start_gather(g):
            slot = g % NSLOTS
            ic = ids_v.at[pl.ds(g * CH, CH)]
            return (
                pltpu.async_copy(wr_hbm.at[ic], bufr[slot], semgr[slot]),
                pltpu.async_copy(wi_hbm.at[ic], bufi[slot], semgi[slot]),
            )

        gath = {}
        outs = {}
        for g in range(min(NSLOTS, n_chunks)):
            gath[g] = start_gather(g)

        for g in range(n_chunks):
            slot = g % NSLOTS
            for cp in gath.pop(g):
                cp.wait()
            dst = pl.ds(tok0 + g * CH, CH)
            outs[g] = (
                pltpu.async_copy(bufr[slot], out_hbm.at[0, dst], semor[slot]),
                pltpu.async_copy(bufi[slot], out_hbm.at[1, dst], semoi[slot]),
            )
            ng = g + NSLOTS
            if ng < n_chunks:
                for cp in outs.pop(g):
                    cp.wait()
                gath[ng] = start_gather(ng)

        for g in sorted(outs):
            for cp in outs[g]:
                cp.wait()

    return gather_kernel


def _rotate_body(theta_ref, g_ref, pos_ref, out_ref):
    b = pl.program_id(0)
    th = jnp.transpose(theta_ref[pl.ds(b, 1), :])
    c = jnp.cos(th)
    s = jnp.sin(th)
    x = g_ref[0] + pos_ref[...]
    y = g_ref[1]
    out_ref[0, 0] = x * c - y * s
    out_ref[1, 0] = x * s + y * c


def _rotate_body_aliased(theta_ref, g_ref, pos_ref, _prev_ref, out_ref):
    _rotate_body(theta_ref, g_ref, pos_ref, out_ref)


def _rotate_chunk(theta, g, pos_table, c, span, bsz, seq, prev):
    in_specs = [
        pl.BlockSpec((bsz, span), lambda b, c=c: (0, c)),
        pl.BlockSpec((2, span, D), lambda b: (0, b, 0)),
        pl.BlockSpec((span, D), lambda b, c=c: (c, 0)),
    ]
    operands = [theta, g, pos_table]
    body = _rotate_body
    aliases = {}
    if prev is not None:
        in_specs.append(pl.BlockSpec(memory_space=pl.ANY))
        operands.append(prev)
        body = _rotate_body_aliased
        aliases = {3: 0}
    return pl.pallas_call(
        body,
        grid=(bsz,),
        in_specs=in_specs,
        out_specs=pl.BlockSpec(
            (2, 1, span, D), lambda b, c=c: (0, b, c, 0)),
        out_shape=jax.ShapeDtypeStruct((2, bsz, seq, D), jnp.float32),
        input_output_aliases=aliases,
    )(*operands)


def kernel(input_ids, initial_phase, W_real, W_imag, pos_table):
    bsz, seq = input_ids.shape
    span = seq // NCHUNK
    ids_flat = input_ids.astype(jnp.int32).reshape(bsz * seq)
    gathered = [
        _make_gather_kernel(bsz, seq, span, c)(ids_flat, W_real, W_imag)
        for c in range(NCHUNK)
    ]
    out = None
    for c in range(NCHUNK):
        out = _rotate_chunk(initial_phase, gathered[c], pos_table, c, span,
                            bsz, seq, out)
    return out
